# Initial kernel scaffold; baseline (speedup 1.0000x reference)
import jax
import jax.numpy as jnp
from jax.experimental import pallas as pl


def kernel(inp, gate, weight1, weight2):
    raise NotImplementedError("write your pallas kernel here")


# speedup vs baseline: 2.2402x; 2.2402x over previous
import functools

import jax
import jax.numpy as jnp
from jax import lax
from jax.experimental import pallas as pl
from jax.experimental.pallas import tpu as pltpu
from jax.experimental.pallas import tpu_sc as plsc

E = 16
D = 1024
H = 4096
O = 1024
N = 8192
T = 256
NTILES = N // T + E
P = NTILES * T

NW = 32


def _make_sc_gather(n_rows, n_words, chunk, dtype):
    per_w = n_rows // NW
    n_chunks = per_w // chunk
    mesh = plsc.VectorSubcoreMesh(core_axis_name="c", subcore_axis_name="s")

    def body(table_hbm, idx_hbm, out_hbm, idx_v, rows_v, sem):
        wid = lax.axis_index("s") * 2 + lax.axis_index("c")
        base = wid * per_w

        def one_chunk(i, carry):
            off = base + i * chunk
            pltpu.sync_copy(idx_hbm.at[pl.ds(off, chunk)], idx_v)
            pltpu.async_copy(table_hbm.at[idx_v], rows_v, sem).wait()
            pltpu.sync_copy(rows_v, out_hbm.at[pl.ds(off, chunk)])
            return carry

        lax.fori_loop(0, n_chunks, one_chunk, 0)

    return functools.partial(
        pl.kernel,
        mesh=mesh,
        out_type=jax.ShapeDtypeStruct((n_rows, n_words), dtype),
        scratch_types=[
            pltpu.VMEM((chunk,), jnp.int32),
            pltpu.VMEM((chunk, n_words), dtype),
            pltpu.SemaphoreType.DMA,
        ],
    )(body)


def _mm_body(em_ref, x_ref, w1_ref, w2_ref, out_ref):
    x = x_ref[...]
    w1 = w1_ref[0]
    w2 = w2_ref[0]
    HC = 1024
    acc = jnp.zeros((T, O), jnp.float32)
    for c in range(H // HC):
        h = lax.dot_general(
            x, w1[c * HC:(c + 1) * HC, :],
            (((1,), (1,)), ((), ())),
            preferred_element_type=jnp.float32,
        )
        acc = acc + lax.dot_general(
            h.astype(jnp.bfloat16), w2[:, c * HC:(c + 1) * HC],
            (((1,), (1,)), ((), ())),
            preferred_element_type=jnp.float32,
        )
    out_ref[...] = acc


def _moe_matmul(x_pad, w1, w2, expert_map):
    grid_spec = pltpu.PrefetchScalarGridSpec(
        num_scalar_prefetch=1,
        grid=(NTILES,),
        in_specs=[
            pl.BlockSpec((T, D), lambda t, em: (t, 0)),
            pl.BlockSpec((1, H, D), lambda t, em: (em[t], 0, 0)),
            pl.BlockSpec((1, O, H), lambda t, em: (em[t], 0, 0)),
        ],
        out_specs=pl.BlockSpec((T, O), lambda t, em: (t, 0)),
    )
    return pl.pallas_call(
        _mm_body,
        grid_spec=grid_spec,
        out_shape=jax.ShapeDtypeStruct((P, O), jnp.float32),
    )(expert_map, x_pad, w1, w2)


def kernel(inp, gate, weight1, weight2):
    gate = gate.astype(jnp.int32)

    sort_idx = jnp.argsort(gate).astype(jnp.int32)
    sorted_gate = gate[sort_idx]
    counts = jnp.zeros((E,), jnp.int32).at[gate].add(1)
    tiles_per_e = (counts + T - 1) // T
    seg_start = jnp.concatenate([jnp.zeros((1,), jnp.int32),
                                 jnp.cumsum(counts)[:-1]]).astype(jnp.int32)
    tile_start = jnp.concatenate([jnp.zeros((1,), jnp.int32),
                                  jnp.cumsum(tiles_per_e)[:-1]]).astype(jnp.int32)
    pad_off = tile_start * T
    s = jnp.arange(N, dtype=jnp.int32)
    pad_pos = pad_off[sorted_gate] + (s - seg_start[sorted_gate])
    src_idx = jnp.zeros((P,), jnp.int32).at[pad_pos].set(sort_idx)
    inv_map = jnp.zeros((N,), jnp.int32).at[sort_idx].set(pad_pos)
    t_ids = jnp.arange(NTILES, dtype=jnp.int32)
    expert_map = jnp.clip(
        jnp.sum(t_ids[:, None] >= tile_start[None, :], axis=1) - 1, 0, E - 1
    ).astype(jnp.int32)

    x_bf = inp.astype(jnp.bfloat16)
    x_words = lax.bitcast_convert_type(
        x_bf.reshape(N, D // 2, 2), jnp.float32)
    xpad_words = _make_sc_gather(P, D // 2, 128, jnp.float32)(x_words, src_idx)
    x_pad = lax.bitcast_convert_type(
        xpad_words, jnp.bfloat16).reshape(P, D)

    w1 = weight1.astype(jnp.bfloat16)
    w2 = weight2.astype(jnp.bfloat16)
    out_pad = _moe_matmul(x_pad, w1, w2, expert_map)

    out = _make_sc_gather(N, O, 64, jnp.float32)(out_pad, inv_map)
    return out


# speedup vs baseline: 2.8926x; 1.2913x over previous
import functools

import jax
import jax.numpy as jnp
from jax import lax
from jax.experimental import pallas as pl
from jax.experimental.pallas import tpu as pltpu
from jax.experimental.pallas import tpu_sc as plsc

E = 16
D = 1024
H = 4096
O = 1024
N = 8192
T = 256
NTILES = N // T + E
P = NTILES * T

NW = 32


def _sc_scatter_rows(table, idx3):
    n_words = table.shape[1]
    k, chunk = idx3.shape[1], idx3.shape[2]
    per_w = k * chunk
    mesh = plsc.VectorSubcoreMesh(core_axis_name="c", subcore_axis_name="s")

    def body(table_hbm, idx_hbm, out_hbm, idx_v, rows_v, sem):
        wid = lax.axis_index("s") * 2 + lax.axis_index("c")
        base = wid * per_w
        pltpu.sync_copy(idx_hbm.at[wid], idx_v)

        def one_chunk(j, carry):
            pltpu.sync_copy(table_hbm.at[pl.ds(base + j * chunk, chunk)], rows_v)
            pltpu.async_copy(rows_v, out_hbm.at[idx_v.at[j]], sem).wait()
            return carry

        lax.fori_loop(0, k, one_chunk, 0)

    return pl.kernel(
        body,
        mesh=mesh,
        out_type=jax.ShapeDtypeStruct((P, n_words), jnp.float32),
        scratch_types=[
            pltpu.VMEM((k, chunk), jnp.int32),
            pltpu.VMEM((chunk, n_words), jnp.float32),
            pltpu.SemaphoreType.DMA,
        ],
    )(table, idx3)


def _sc_gather_rows(table, idx, chunk):
    n_rows = idx.shape[0]
    n_words = table.shape[1]
    per_w = n_rows // NW
    n_chunks = per_w // chunk
    mesh = plsc.VectorSubcoreMesh(core_axis_name="c", subcore_axis_name="s")

    def body(table_hbm, idx_hbm, out_hbm, idx_v, rows_v, sem):
        wid = lax.axis_index("s") * 2 + lax.axis_index("c")
        base = wid * per_w

        def one_chunk(i, carry):
            off = base + i * chunk
            pltpu.sync_copy(idx_hbm.at[pl.ds(off, chunk)], idx_v)
            pltpu.async_copy(table_hbm.at[idx_v], rows_v, sem).wait()
            pltpu.sync_copy(rows_v, out_hbm.at[pl.ds(off, chunk)])
            return carry

        lax.fori_loop(0, n_chunks, one_chunk, 0)

    return pl.kernel(
        body,
        mesh=mesh,
        out_type=jax.ShapeDtypeStruct((n_rows, n_words), jnp.float32),
        scratch_types=[
            pltpu.VMEM((chunk,), jnp.int32),
            pltpu.VMEM((chunk, n_words), jnp.float32),
            pltpu.SemaphoreType.DMA,
        ],
    )(table, idx)


def _mm_body(em_ref, nt_ref, x_ref, w1_ref, w2_ref, out_ref):
    t = pl.program_id(0)

    @pl.when(t < nt_ref[0])
    def _():
        x = x_ref[...]
        w1 = w1_ref[0]
        w2 = w2_ref[0]
        HC = 1024
        acc = jnp.zeros((T, O), jnp.float32)
        for c in range(H // HC):
            h = lax.dot_general(
                x, w1[c * HC:(c + 1) * HC, :],
                (((1,), (1,)), ((), ())),
                preferred_element_type=jnp.float32,
            )
            acc = acc + lax.dot_general(
                h.astype(jnp.bfloat16), w2[:, c * HC:(c + 1) * HC],
                (((1,), (1,)), ((), ())),
                preferred_element_type=jnp.float32,
            )
        out_ref[...] = acc


def _moe_matmul(x_pad, w1, w2, expert_map, n_tiles):
    grid_spec = pltpu.PrefetchScalarGridSpec(
        num_scalar_prefetch=2,
        grid=(NTILES,),
        in_specs=[
            pl.BlockSpec((T, D), lambda t, em, nt: (t, 0)),
            pl.BlockSpec((1, H, D), lambda t, em, nt: (em[t], 0, 0)),
            pl.BlockSpec((1, O, H), lambda t, em, nt: (em[t], 0, 0)),
        ],
        out_specs=pl.BlockSpec((T, O), lambda t, em, nt: (t, 0)),
    )
    return pl.pallas_call(
        _mm_body,
        grid_spec=grid_spec,
        out_shape=jax.ShapeDtypeStruct((P, O), jnp.float32),
    )(expert_map, n_tiles, x_pad, w1, w2)


def kernel(inp, gate, weight1, weight2):
    gate = gate.astype(jnp.int32)

    onehot = (gate[:, None] == jnp.arange(E, dtype=jnp.int32)[None, :])
    onehot_i = onehot.astype(jnp.int32)
    incl = jnp.cumsum(onehot_i, axis=0)
    counts = incl[-1]
    rank = jnp.sum(jnp.where(onehot, incl, 0), axis=1) - 1
    tiles_per_e = (counts + T - 1) // T
    tile_start = jnp.concatenate([jnp.zeros((1,), jnp.int32),
                                  jnp.cumsum(tiles_per_e)[:-1]]).astype(jnp.int32)
    n_tiles = tile_start[-1] + tiles_per_e[-1]
    pad_off = tile_start * T
    pad_pos = jnp.sum(jnp.where(onehot, pad_off[None, :], 0), axis=1) + rank
    pad_pos = pad_pos.astype(jnp.int32)
    t_ids = jnp.arange(NTILES, dtype=jnp.int32)
    expert_map = jnp.clip(
        jnp.sum((t_ids[:, None] >= tile_start[None, :]).astype(jnp.int32),
                axis=1) - 1, 0, E - 1).astype(jnp.int32)

    x_bf = inp.astype(jnp.bfloat16)
    x_words = lax.bitcast_convert_type(
        x_bf.reshape(N, D // 2, 2), jnp.float32)
    idx3 = pad_pos.reshape(NW, 2, N // NW // 2)
    xpad_words = _sc_scatter_rows(x_words, idx3)
    x_pad = lax.bitcast_convert_type(
        xpad_words, jnp.bfloat16).reshape(P, D)

    w1 = weight1.astype(jnp.bfloat16)
    w2 = weight2.astype(jnp.bfloat16)
    out_pad = _moe_matmul(x_pad, w1, w2, expert_map,
                          n_tiles.reshape(1))

    return _sc_gather_rows(out_pad, pad_pos, 64)


# speedup vs baseline: 3.2288x; 1.1162x over previous
import functools

import jax
import jax.numpy as jnp
from jax import lax
from jax.experimental import pallas as pl
from jax.experimental.pallas import tpu as pltpu
from jax.experimental.pallas import tpu_sc as plsc

E = 16
D = 1024
H = 4096
O = 1024
N = 8192
T = 256
NTILES = N // T + E
P = NTILES * T

NW = 32


def _sc_scatter_rows(table, idx3):
    n_words = table.shape[1]
    k, chunk = idx3.shape[1], idx3.shape[2]
    per_w = k * chunk
    mesh = plsc.VectorSubcoreMesh(core_axis_name="c", subcore_axis_name="s")

    def body(table_hbm, idx_hbm, out_hbm, idx_v, rows_v, sem):
        wid = lax.axis_index("s") * 2 + lax.axis_index("c")
        base = wid * per_w
        pltpu.sync_copy(idx_hbm.at[wid], idx_v)

        def one_chunk(j, carry):
            pltpu.sync_copy(table_hbm.at[pl.ds(base + j * chunk, chunk)], rows_v)
            pltpu.async_copy(rows_v, out_hbm.at[idx_v.at[j]], sem).wait()
            return carry

        lax.fori_loop(0, k, one_chunk, 0)

    return pl.kernel(
        body,
        mesh=mesh,
        out_type=jax.ShapeDtypeStruct((P, n_words), jnp.float32),
        scratch_types=[
            pltpu.VMEM((k, chunk), jnp.int32),
            pltpu.VMEM((chunk, n_words), jnp.float32),
            pltpu.SemaphoreType.DMA,
        ],
    )(table, idx3)


def _sc_gather_rows(table, idx, chunk):
    n_rows = idx.shape[0]
    n_words = table.shape[1]
    per_w = n_rows // NW
    n_chunks = per_w // chunk
    mesh = plsc.VectorSubcoreMesh(core_axis_name="c", subcore_axis_name="s")

    def body(table_hbm, idx_hbm, out_hbm, idx_v, rows_v, sem):
        wid = lax.axis_index("s") * 2 + lax.axis_index("c")
        base = wid * per_w

        def one_chunk(i, carry):
            off = base + i * chunk
            pltpu.sync_copy(idx_hbm.at[pl.ds(off, chunk)], idx_v)
            pltpu.async_copy(table_hbm.at[idx_v], rows_v, sem).wait()
            pltpu.sync_copy(rows_v, out_hbm.at[pl.ds(off, chunk)])
            return carry

        lax.fori_loop(0, n_chunks, one_chunk, 0)

    return pl.kernel(
        body,
        mesh=mesh,
        out_type=jax.ShapeDtypeStruct((n_rows, n_words), jnp.float32),
        scratch_types=[
            pltpu.VMEM((chunk,), jnp.int32),
            pltpu.VMEM((chunk, n_words), jnp.float32),
            pltpu.SemaphoreType.DMA,
        ],
    )(table, idx)


def _layer_body(em_ref, nt_ref, x_ref, w_ref, out_ref, out_dtype):
    t = pl.program_id(0)

    @pl.when(t < nt_ref[0])
    def _():
        x = x_ref[...].astype(jnp.float32)
        out_ref[...] = lax.dot_general(
            x, w_ref[0],
            (((1,), (1,)), ((), ())),
            preferred_element_type=jnp.float32,
        ).astype(out_dtype)


def _moe_layer(x_pad, w, expert_map, n_tiles, out_dtype):
    _, M, K = w.shape
    grid_spec = pltpu.PrefetchScalarGridSpec(
        num_scalar_prefetch=2,
        grid=(NTILES,),
        in_specs=[
            pl.BlockSpec((T, K), lambda t, em, nt: (t, 0)),
            pl.BlockSpec((1, M, K), lambda t, em, nt: (em[t], 0, 0)),
        ],
        out_specs=pl.BlockSpec((T, M), lambda t, em, nt: (t, 0)),
    )
    return pl.pallas_call(
        functools.partial(_layer_body, out_dtype=out_dtype),
        grid_spec=grid_spec,
        out_shape=jax.ShapeDtypeStruct((P, M), out_dtype),
        compiler_params=pltpu.CompilerParams(
            vmem_limit_bytes=56 * 1024 * 1024),
    )(expert_map, n_tiles, x_pad, w)


def kernel(inp, gate, weight1, weight2):
    gate = gate.astype(jnp.int32)

    onehot = (gate[:, None] == jnp.arange(E, dtype=jnp.int32)[None, :])
    onehot_i = onehot.astype(jnp.int32)
    incl = jnp.cumsum(onehot_i, axis=0)
    counts = incl[-1]
    rank = jnp.sum(jnp.where(onehot, incl, 0), axis=1) - 1
    tiles_per_e = (counts + T - 1) // T
    tile_start = jnp.concatenate([jnp.zeros((1,), jnp.int32),
                                  jnp.cumsum(tiles_per_e)[:-1]]).astype(jnp.int32)
    n_tiles = tile_start[-1] + tiles_per_e[-1]
    pad_off = tile_start * T
    pad_pos = jnp.sum(jnp.where(onehot, pad_off[None, :], 0), axis=1) + rank
    pad_pos = pad_pos.astype(jnp.int32)
    t_ids = jnp.arange(NTILES, dtype=jnp.int32)
    expert_map = jnp.clip(
        jnp.sum((t_ids[:, None] >= tile_start[None, :]).astype(jnp.int32),
                axis=1) - 1, 0, E - 1).astype(jnp.int32)

    x_bf = inp.astype(jnp.bfloat16)
    x_words = lax.bitcast_convert_type(
        x_bf.reshape(N, D // 2, 2), jnp.float32)
    idx3 = pad_pos.reshape(NW, 2, N // NW // 2)
    xpad_words = _sc_scatter_rows(x_words, idx3)
    x_pad = lax.bitcast_convert_type(
        xpad_words, jnp.bfloat16).reshape(P, D)

    nt = n_tiles.reshape(1)
    h_pad = _moe_layer(x_pad, weight1, expert_map, nt, jnp.bfloat16)
    out_pad = _moe_layer(h_pad, weight2, expert_map, nt, jnp.float32)

    return _sc_gather_rows(out_pad, pad_pos, 64)


# speedup vs baseline: 3.2307x; 1.0006x over previous
import functools

import jax
import jax.numpy as jnp
from jax import lax
from jax.experimental import pallas as pl
from jax.experimental.pallas import tpu as pltpu
from jax.experimental.pallas import tpu_sc as plsc

E = 16
D = 1024
H = 4096
O = 1024
N = 8192
T = 256
NTILES = N // T + E
P = NTILES * T

NW = 32


def _sc_scatter_rows(table, idx3):
    n_words = table.shape[1]
    k, chunk = idx3.shape[1], idx3.shape[2]
    per_w = k * chunk
    mesh = plsc.VectorSubcoreMesh(core_axis_name="c", subcore_axis_name="s")

    def body(table_hbm, idx_hbm, out_hbm, idx_v, rows_v, sem):
        wid = lax.axis_index("s") * 2 + lax.axis_index("c")
        base = wid * per_w
        pltpu.sync_copy(idx_hbm.at[wid], idx_v)

        def one_chunk(j, carry):
            pltpu.sync_copy(table_hbm.at[pl.ds(base + j * chunk, chunk)], rows_v)
            pltpu.async_copy(rows_v, out_hbm.at[idx_v.at[j]], sem).wait()
            return carry

        lax.fori_loop(0, k, one_chunk, 0)

    return pl.kernel(
        body,
        mesh=mesh,
        out_type=jax.ShapeDtypeStruct((P, n_words), jnp.float32),
        scratch_types=[
            pltpu.VMEM((k, chunk), jnp.int32),
            pltpu.VMEM((chunk, n_words), jnp.float32),
            pltpu.SemaphoreType.DMA,
        ],
        compiler_params=pltpu.CompilerParams(use_tc_tiling_on_sc=True),
    )(table, idx3)


def _sc_gather_rows(table, idx, chunk):
    n_rows = idx.shape[0]
    n_words = table.shape[1]
    per_w = n_rows // NW
    n_chunks = per_w // chunk
    mesh = plsc.VectorSubcoreMesh(core_axis_name="c", subcore_axis_name="s")

    def body(table_hbm, idx_hbm, out_hbm, idx_v, rows_v, sem):
        wid = lax.axis_index("s") * 2 + lax.axis_index("c")
        base = wid * per_w

        def one_chunk(i, carry):
            off = base + i * chunk
            pltpu.sync_copy(idx_hbm.at[pl.ds(off, chunk)], idx_v)
            pltpu.async_copy(table_hbm.at[idx_v], rows_v, sem).wait()
            pltpu.sync_copy(rows_v, out_hbm.at[pl.ds(off, chunk)])
            return carry

        lax.fori_loop(0, n_chunks, one_chunk, 0)

    return pl.kernel(
        body,
        mesh=mesh,
        out_type=jax.ShapeDtypeStruct((n_rows, n_words), jnp.float32),
        scratch_types=[
            pltpu.VMEM((chunk,), jnp.int32),
            pltpu.VMEM((chunk, n_words), jnp.float32),
            pltpu.SemaphoreType.DMA,
        ],
        compiler_params=pltpu.CompilerParams(use_tc_tiling_on_sc=True),
    )(table, idx)


def _layer_body(em_ref, nt_ref, x_ref, w_ref, out_ref, out_dtype):
    t = pl.program_id(0)

    @pl.when(t < nt_ref[0])
    def _():
        x = x_ref[...].astype(jnp.float32)
        out_ref[...] = lax.dot_general(
            x, w_ref[0],
            (((1,), (1,)), ((), ())),
            preferred_element_type=jnp.float32,
        ).astype(out_dtype)


def _moe_layer(x_pad, w, expert_map, n_tiles, out_dtype):
    _, M, K = w.shape
    grid_spec = pltpu.PrefetchScalarGridSpec(
        num_scalar_prefetch=2,
        grid=(NTILES,),
        in_specs=[
            pl.BlockSpec((T, K), lambda t, em, nt: (t, 0)),
            pl.BlockSpec((1, M, K), lambda t, em, nt: (em[t], 0, 0)),
        ],
        out_specs=pl.BlockSpec((T, M), lambda t, em, nt: (t, 0)),
    )
    return pl.pallas_call(
        functools.partial(_layer_body, out_dtype=out_dtype),
        grid_spec=grid_spec,
        out_shape=jax.ShapeDtypeStruct((P, M), out_dtype),
        compiler_params=pltpu.CompilerParams(
            vmem_limit_bytes=56 * 1024 * 1024),
    )(expert_map, n_tiles, x_pad, w)


def kernel(inp, gate, weight1, weight2):
    gate = gate.astype(jnp.int32)

    onehot = (gate[:, None] == jnp.arange(E, dtype=jnp.int32)[None, :])
    onehot_i = onehot.astype(jnp.int32)
    incl = jnp.cumsum(onehot_i, axis=0)
    counts = incl[-1]
    rank = jnp.sum(jnp.where(onehot, incl, 0), axis=1) - 1
    tiles_per_e = (counts + T - 1) // T
    tile_start = jnp.concatenate([jnp.zeros((1,), jnp.int32),
                                  jnp.cumsum(tiles_per_e)[:-1]]).astype(jnp.int32)
    n_tiles = tile_start[-1] + tiles_per_e[-1]
    pad_off = tile_start * T
    pad_pos = jnp.sum(jnp.where(onehot, pad_off[None, :], 0), axis=1) + rank
    pad_pos = pad_pos.astype(jnp.int32)
    t_ids = jnp.arange(NTILES, dtype=jnp.int32)
    expert_map = jnp.clip(
        jnp.sum((t_ids[:, None] >= tile_start[None, :]).astype(jnp.int32),
                axis=1) - 1, 0, E - 1).astype(jnp.int32)

    x_bf = inp.astype(jnp.bfloat16)
    x_words = lax.bitcast_convert_type(
        x_bf.reshape(N, D // 2, 2), jnp.float32)
    idx3 = pad_pos.reshape(NW, 2, N // NW // 2)
    xpad_words = _sc_scatter_rows(x_words, idx3)
    x_pad = lax.bitcast_convert_type(
        xpad_words, jnp.bfloat16).reshape(P, D)

    nt = n_tiles.reshape(1)
    h_pad = _moe_layer(x_pad, weight1, expert_map, nt, jnp.bfloat16)
    out_pad = _moe_layer(h_pad, weight2, expert_map, nt, jnp.float32)

    return _sc_gather_rows(out_pad, pad_pos, 64)


# speedup vs baseline: 6.0494x; 1.8725x over previous
import functools

import jax
import jax.numpy as jnp
from jax import lax
from jax.experimental import pallas as pl
from jax.experimental.pallas import tpu as pltpu
from jax.experimental.pallas import tpu_sc as plsc

E = 16
D = 1024
H = 4096
O = 1024
N = 8192
T = 256
NTILES = N // T + E
P = NTILES * T

NW = 32


def _sc_scatter_rows(table, idx3):
    n_words = table.shape[1]
    k, chunk = idx3.shape[1], idx3.shape[2]
    per_w = k * chunk
    mesh = plsc.VectorSubcoreMesh(core_axis_name="c", subcore_axis_name="s")

    def body(table_hbm, idx_hbm, out_hbm, idx_v, rows_v, sem):
        wid = lax.axis_index("s") * 2 + lax.axis_index("c")
        base = wid * per_w
        pltpu.sync_copy(idx_hbm.at[wid], idx_v)

        def one_chunk(j, carry):
            pltpu.sync_copy(table_hbm.at[pl.ds(base + j * chunk, chunk)], rows_v)
            pltpu.async_copy(rows_v, out_hbm.at[idx_v.at[j]], sem).wait()
            return carry

        lax.fori_loop(0, k, one_chunk, 0)

    return pl.kernel(
        body,
        mesh=mesh,
        out_type=jax.ShapeDtypeStruct((P, n_words), jnp.float32),
        scratch_types=[
            pltpu.VMEM((k, chunk), jnp.int32),
            pltpu.VMEM((chunk, n_words), jnp.float32),
            pltpu.SemaphoreType.DMA,
        ],
    )(table, idx3)


def _sc_gather_rows(table, idx, chunk):
    n_rows = idx.shape[0]
    n_words = table.shape[1]
    per_w = n_rows // NW
    n_chunks = per_w // chunk
    mesh = plsc.VectorSubcoreMesh(core_axis_name="c", subcore_axis_name="s")

    def body(table_hbm, idx_hbm, out_hbm, idx_v, rows_v, sem):
        wid = lax.axis_index("s") * 2 + lax.axis_index("c")
        base = wid * per_w

        def one_chunk(i, carry):
            off = base + i * chunk
            pltpu.sync_copy(idx_hbm.at[pl.ds(off, chunk)], idx_v)
            pltpu.async_copy(table_hbm.at[idx_v], rows_v, sem).wait()
            pltpu.sync_copy(rows_v, out_hbm.at[pl.ds(off, chunk)])
            return carry

        lax.fori_loop(0, n_chunks, one_chunk, 0)

    return pl.kernel(
        body,
        mesh=mesh,
        out_type=jax.ShapeDtypeStruct((n_rows, n_words), jnp.float32),
        scratch_types=[
            pltpu.VMEM((chunk,), jnp.int32),
            pltpu.VMEM((chunk, n_words), jnp.float32),
            pltpu.SemaphoreType.DMA,
        ],
    )(table, idx)


def _layer_body(em_ref, nt_ref, x_ref, w_ref, out_ref, out_dtype):
    t = pl.program_id(0)

    @pl.when(t < nt_ref[0])
    def _():
        x = x_ref[...].astype(jnp.float32)
        out_ref[...] = lax.dot_general(
            x, w_ref[0],
            (((1,), (1,)), ((), ())),
            preferred_element_type=jnp.float32,
        ).astype(out_dtype)


def _moe_layer(x_pad, w, expert_map, n_tiles, out_dtype):
    _, M, K = w.shape
    grid_spec = pltpu.PrefetchScalarGridSpec(
        num_scalar_prefetch=2,
        grid=(NTILES,),
        in_specs=[
            pl.BlockSpec((T, K), lambda t, em, nt: (t, 0)),
            pl.BlockSpec((1, M, K), lambda t, em, nt: (em[t], 0, 0)),
        ],
        out_specs=pl.BlockSpec((T, M), lambda t, em, nt: (t, 0)),
    )
    return pl.pallas_call(
        functools.partial(_layer_body, out_dtype=out_dtype),
        grid_spec=grid_spec,
        out_shape=jax.ShapeDtypeStruct((P, M), out_dtype),
        compiler_params=pltpu.CompilerParams(
            vmem_limit_bytes=56 * 1024 * 1024),
    )(expert_map, n_tiles, x_pad, w)


def kernel(inp, gate, weight1, weight2):
    gate = gate.astype(jnp.int32)

    onehot = (gate[:, None] == jnp.arange(E, dtype=jnp.int32)[None, :])
    onehot_i = onehot.astype(jnp.int32)
    incl = jnp.cumsum(onehot_i, axis=0)
    counts = incl[-1]
    rank = jnp.sum(jnp.where(onehot, incl, 0), axis=1) - 1
    tiles_per_e = (counts + T - 1) // T
    tile_start = jnp.concatenate([jnp.zeros((1,), jnp.int32),
                                  jnp.cumsum(tiles_per_e)[:-1]]).astype(jnp.int32)
    n_tiles = tile_start[-1] + tiles_per_e[-1]
    pad_off = tile_start * T
    pad_pos = jnp.sum(jnp.where(onehot, pad_off[None, :], 0), axis=1) + rank
    pad_pos = pad_pos.astype(jnp.int32)
    t_ids = jnp.arange(NTILES, dtype=jnp.int32)
    expert_map = jnp.clip(
        jnp.sum((t_ids[:, None] >= tile_start[None, :]).astype(jnp.int32),
                axis=1) - 1, 0, E - 1).astype(jnp.int32)

    idx3 = pad_pos.reshape(NW, 4, N // NW // 4)
    x_pad = _sc_scatter_rows(inp, idx3)

    nt = n_tiles.reshape(1)
    h_pad = _moe_layer(x_pad, weight1, expert_map, nt, jnp.bfloat16)
    out_pad = _moe_layer(h_pad, weight2, expert_map, nt, jnp.float32)

    return _sc_gather_rows(out_pad, pad_pos, 64)


# speedup vs baseline: 6.3589x; 1.0511x over previous
import functools

import jax
import jax.numpy as jnp
from jax import lax
from jax.experimental import pallas as pl
from jax.experimental.pallas import tpu as pltpu
from jax.experimental.pallas import tpu_sc as plsc

E = 16
D = 1024
H = 4096
O = 1024
N = 8192
T = 512
NTILES = N // T + E
P = NTILES * T

NW = 32


def _sc_scatter_rows(table, idx3):
    n_words = table.shape[1]
    k, chunk = idx3.shape[1], idx3.shape[2]
    per_w = k * chunk
    mesh = plsc.VectorSubcoreMesh(core_axis_name="c", subcore_axis_name="s")

    def body(table_hbm, idx_hbm, out_hbm, idx_v, rows_v, sem):
        wid = lax.axis_index("s") * 2 + lax.axis_index("c")
        base = wid * per_w
        pltpu.sync_copy(idx_hbm.at[wid], idx_v)

        def one_chunk(j, carry):
            pltpu.sync_copy(table_hbm.at[pl.ds(base + j * chunk, chunk)], rows_v)
            pltpu.async_copy(rows_v, out_hbm.at[idx_v.at[j]], sem).wait()
            return carry

        lax.fori_loop(0, k, one_chunk, 0)

    return pl.kernel(
        body,
        mesh=mesh,
        out_type=jax.ShapeDtypeStruct((P, n_words), jnp.float32),
        scratch_types=[
            pltpu.VMEM((k, chunk), jnp.int32),
            pltpu.VMEM((chunk, n_words), jnp.float32),
            pltpu.SemaphoreType.DMA,
        ],
    )(table, idx3)


def _sc_gather_rows(table, idx, chunk):
    n_rows = idx.shape[0]
    n_words = table.shape[1]
    per_w = n_rows // NW
    n_chunks = per_w // chunk
    mesh = plsc.VectorSubcoreMesh(core_axis_name="c", subcore_axis_name="s")

    def body(table_hbm, idx_hbm, out_hbm, idx_v, rows_v, sem):
        wid = lax.axis_index("s") * 2 + lax.axis_index("c")
        base = wid * per_w

        def one_chunk(i, carry):
            off = base + i * chunk
            pltpu.sync_copy(idx_hbm.at[pl.ds(off, chunk)], idx_v)
            pltpu.async_copy(table_hbm.at[idx_v], rows_v, sem).wait()
            pltpu.sync_copy(rows_v, out_hbm.at[pl.ds(off, chunk)])
            return carry

        lax.fori_loop(0, n_chunks, one_chunk, 0)

    return pl.kernel(
        body,
        mesh=mesh,
        out_type=jax.ShapeDtypeStruct((n_rows, n_words), jnp.float32),
        scratch_types=[
            pltpu.VMEM((chunk,), jnp.int32),
            pltpu.VMEM((chunk, n_words), jnp.float32),
            pltpu.SemaphoreType.DMA,
        ],
    )(table, idx)


def _layer_body(em_ref, nt_ref, x_ref, w_ref, out_ref, out_dtype):
    t = pl.program_id(0)

    @pl.when(t < nt_ref[0])
    def _():
        x = x_ref[...].astype(jnp.float32)
        out_ref[...] = lax.dot_general(
            x, w_ref[0],
            (((1,), (1,)), ((), ())),
            preferred_element_type=jnp.float32,
        ).astype(out_dtype)


def _moe_layer(x_pad, w, expert_map, n_tiles, out_dtype):
    _, M, K = w.shape
    grid_spec = pltpu.PrefetchScalarGridSpec(
        num_scalar_prefetch=2,
        grid=(NTILES,),
        in_specs=[
            pl.BlockSpec((T, K), lambda t, em, nt: (jnp.minimum(t, nt[0] - 1), 0)),
            pl.BlockSpec((1, M, K), lambda t, em, nt: (em[t], 0, 0)),
        ],
        out_specs=pl.BlockSpec((T, M), lambda t, em, nt: (t, 0)),
    )
    return pl.pallas_call(
        functools.partial(_layer_body, out_dtype=out_dtype),
        grid_spec=grid_spec,
        out_shape=jax.ShapeDtypeStruct((P, M), out_dtype),
        compiler_params=pltpu.CompilerParams(
            vmem_limit_bytes=56 * 1024 * 1024),
    )(expert_map, n_tiles, x_pad, w)


def kernel(inp, gate, weight1, weight2):
    gate = gate.astype(jnp.int32)

    onehot = (gate[:, None] == jnp.arange(E, dtype=jnp.int32)[None, :])
    onehot_i = onehot.astype(jnp.int32)
    incl = jnp.cumsum(onehot_i, axis=0)
    counts = incl[-1]
    rank = jnp.sum(jnp.where(onehot, incl, 0), axis=1) - 1
    tiles_per_e = (counts + T - 1) // T
    tile_start = jnp.concatenate([jnp.zeros((1,), jnp.int32),
                                  jnp.cumsum(tiles_per_e)[:-1]]).astype(jnp.int32)
    n_tiles = tile_start[-1] + tiles_per_e[-1]
    pad_off = tile_start * T
    pad_pos = jnp.sum(jnp.where(onehot, pad_off[None, :], 0), axis=1) + rank
    pad_pos = pad_pos.astype(jnp.int32)
    t_ids = jnp.arange(NTILES, dtype=jnp.int32)
    expert_map = jnp.clip(
        jnp.sum((t_ids[:, None] >= tile_start[None, :]).astype(jnp.int32),
                axis=1) - 1, 0, E - 1).astype(jnp.int32)

    idx3 = pad_pos.reshape(NW, 4, N // NW // 4)
    x_pad = _sc_scatter_rows(inp, idx3)

    nt = n_tiles.reshape(1)
    h_pad = _moe_layer(x_pad, weight1, expert_map, nt, jnp.bfloat16)
    out_pad = _moe_layer(h_pad, weight2, expert_map, nt, jnp.float32)

    return _sc_gather_rows(out_pad, pad_pos, 64)


# speedup vs baseline: 6.6793x; 1.0504x over previous
import functools

import jax
import jax.numpy as jnp
from jax import lax
from jax.experimental import pallas as pl
from jax.experimental.pallas import tpu as pltpu
from jax.experimental.pallas import tpu_sc as plsc

E = 16
D = 1024
H = 4096
O = 1024
N = 8192
T = 512
NTILES = N // T + E
P = NTILES * T

NW = 32


def _sc_scatter_rows(table, idx3):
    n_words = table.shape[1]
    k, chunk = idx3.shape[1], idx3.shape[2]
    per_w = k * chunk
    mesh = plsc.VectorSubcoreMesh(core_axis_name="c", subcore_axis_name="s")

    def body(table_hbm, idx_hbm, out_hbm, idx_v, rows_v, sem):
        wid = lax.axis_index("s") * 2 + lax.axis_index("c")
        base = wid * per_w
        pltpu.sync_copy(idx_hbm.at[wid], idx_v)

        def one_chunk(j, carry):
            pltpu.sync_copy(table_hbm.at[pl.ds(base + j * chunk, chunk)], rows_v)
            pltpu.async_copy(rows_v, out_hbm.at[idx_v.at[j]], sem).wait()
            return carry

        lax.fori_loop(0, k, one_chunk, 0)

    return pl.kernel(
        body,
        mesh=mesh,
        out_type=jax.ShapeDtypeStruct((P, n_words), jnp.float32),
        scratch_types=[
            pltpu.VMEM((k, chunk), jnp.int32),
            pltpu.VMEM((chunk, n_words), jnp.float32),
            pltpu.SemaphoreType.DMA,
        ],
    )(table, idx3)


def _sc_gather_rows(table, idx, chunk):
    n_rows = idx.shape[0]
    n_words = table.shape[1]
    per_w = n_rows // NW
    n_chunks = per_w // chunk
    mesh = plsc.VectorSubcoreMesh(core_axis_name="c", subcore_axis_name="s")

    def body(table_hbm, idx_hbm, out_hbm, idx_v, rows_v, sem):
        wid = lax.axis_index("s") * 2 + lax.axis_index("c")
        base = wid * per_w

        def one_chunk(i, carry):
            off = base + i * chunk
            pltpu.sync_copy(idx_hbm.at[pl.ds(off, chunk)], idx_v)
            pltpu.async_copy(table_hbm.at[idx_v], rows_v, sem).wait()
            pltpu.sync_copy(rows_v, out_hbm.at[pl.ds(off, chunk)])
            return carry

        lax.fori_loop(0, n_chunks, one_chunk, 0)

    return pl.kernel(
        body,
        mesh=mesh,
        out_type=jax.ShapeDtypeStruct((n_rows, n_words), jnp.float32),
        scratch_types=[
            pltpu.VMEM((chunk,), jnp.int32),
            pltpu.VMEM((chunk, n_words), jnp.float32),
            pltpu.SemaphoreType.DMA,
        ],
    )(table, idx)


NC = 4
HCC = H // NC


def _fused_body(em_ref, nt_ref, x_ref, w1_ref, w2_ref, out_ref):
    t = pl.program_id(0)
    c = pl.program_id(1)

    @pl.when(t < nt_ref[0])
    def _():
        h = lax.dot_general(
            x_ref[...], w1_ref[0],
            (((1,), (1,)), ((), ())),
            preferred_element_type=jnp.float32,
        )
        part = lax.dot_general(
            h, w2_ref[0],
            (((1,), (1,)), ((), ())),
            preferred_element_type=jnp.float32,
        )

        @pl.when(c == 0)
        def _():
            out_ref[...] = part

        @pl.when(c > 0)
        def _():
            out_ref[...] += part


def _c_eff(t, c, nt):
    zig = jnp.where(t % 2 == 0, c, NC - 1 - c)
    frozen = jnp.where((nt - 1) % 2 == 0, NC - 1, 0)
    return jnp.where(t < nt, zig, frozen)


def _moe_fused(x_pad, w1, w2, expert_map, n_tiles):
    grid_spec = pltpu.PrefetchScalarGridSpec(
        num_scalar_prefetch=2,
        grid=(NTILES, NC),
        in_specs=[
            pl.BlockSpec(
                (T, D), lambda t, c, em, nt: (jnp.minimum(t, nt[0] - 1), 0)),
            pl.BlockSpec(
                (1, HCC, D),
                lambda t, c, em, nt: (em[t], _c_eff(t, c, nt[0]), 0)),
            pl.BlockSpec(
                (1, O, HCC),
                lambda t, c, em, nt: (em[t], 0, _c_eff(t, c, nt[0]))),
        ],
        out_specs=pl.BlockSpec((T, O), lambda t, c, em, nt: (t, 0)),
    )
    return pl.pallas_call(
        _fused_body,
        grid_spec=grid_spec,
        out_shape=jax.ShapeDtypeStruct((P, O), jnp.float32),
        compiler_params=pltpu.CompilerParams(
            vmem_limit_bytes=56 * 1024 * 1024),
    )(expert_map, n_tiles, x_pad, w1, w2)


def kernel(inp, gate, weight1, weight2):
    gate = gate.astype(jnp.int32)

    onehot = (gate[:, None] == jnp.arange(E, dtype=jnp.int32)[None, :])
    onehot_i = onehot.astype(jnp.int32)
    incl = jnp.cumsum(onehot_i, axis=0)
    counts = incl[-1]
    rank = jnp.sum(jnp.where(onehot, incl, 0), axis=1) - 1
    tiles_per_e = (counts + T - 1) // T
    tile_start = jnp.concatenate([jnp.zeros((1,), jnp.int32),
                                  jnp.cumsum(tiles_per_e)[:-1]]).astype(jnp.int32)
    n_tiles = tile_start[-1] + tiles_per_e[-1]
    pad_off = tile_start * T
    pad_pos = jnp.sum(jnp.where(onehot, pad_off[None, :], 0), axis=1) + rank
    pad_pos = pad_pos.astype(jnp.int32)
    t_ids = jnp.arange(NTILES, dtype=jnp.int32)
    expert_map = jnp.clip(
        jnp.sum((t_ids[:, None] >= tile_start[None, :]).astype(jnp.int32),
                axis=1) - 1, 0, E - 1).astype(jnp.int32)
    expert_map = expert_map[jnp.minimum(t_ids, n_tiles - 1)]

    idx3 = pad_pos.reshape(NW, 4, N // NW // 4)
    x_pad = _sc_scatter_rows(inp, idx3)

    nt = n_tiles.reshape(1)
    out_pad = _moe_fused(x_pad, weight1, weight2, expert_map, nt)

    return _sc_gather_rows(out_pad, pad_pos, 64)


# speedup vs baseline: 8.3504x; 1.2502x over previous
import functools

import jax
import jax.numpy as jnp
from jax import lax
from jax.experimental import pallas as pl
from jax.experimental.pallas import tpu as pltpu
from jax.experimental.pallas import tpu_sc as plsc

E = 16
D = 1024
H = 4096
O = 1024
N = 8192
T = 640
NTILES = (N + T - 1) // T + E - 1
P = NTILES * T

NW = 32


def _sc_scatter_rows(table, idx3):
    n_words = table.shape[1]
    k, chunk = idx3.shape[1], idx3.shape[2]
    per_w = k * chunk
    mesh = plsc.VectorSubcoreMesh(core_axis_name="c", subcore_axis_name="s")

    def body(table_hbm, idx_hbm, out_hbm, idx_v, rows_v, sem):
        wid = lax.axis_index("s") * 2 + lax.axis_index("c")
        base = wid * per_w
        pltpu.sync_copy(idx_hbm.at[wid], idx_v)

        def one_chunk(j, carry):
            pltpu.sync_copy(table_hbm.at[pl.ds(base + j * chunk, chunk)], rows_v)
            pltpu.async_copy(rows_v, out_hbm.at[idx_v.at[j]], sem).wait()
            return carry

        lax.fori_loop(0, k, one_chunk, 0)

    return pl.kernel(
        body,
        mesh=mesh,
        out_type=jax.ShapeDtypeStruct((P, n_words), jnp.float32),
        scratch_types=[
            pltpu.VMEM((k, chunk), jnp.int32),
            pltpu.VMEM((chunk, n_words), jnp.float32),
            pltpu.SemaphoreType.DMA,
        ],
    )(table, idx3)


def _sc_gather_rows(table, idx, chunk):
    n_rows = idx.shape[0]
    n_words = table.shape[1]
    per_w = n_rows // NW
    n_chunks = per_w // chunk
    mesh = plsc.VectorSubcoreMesh(core_axis_name="c", subcore_axis_name="s")

    def body(table_hbm, idx_hbm, out_hbm, idx_v, rows_v, sem):
        wid = lax.axis_index("s") * 2 + lax.axis_index("c")
        base = wid * per_w

        def one_chunk(i, carry):
            off = base + i * chunk
            pltpu.sync_copy(idx_hbm.at[pl.ds(off, chunk)], idx_v)
            pltpu.async_copy(table_hbm.at[idx_v], rows_v, sem).wait()
            pltpu.sync_copy(rows_v, out_hbm.at[pl.ds(off, chunk)])
            return carry

        lax.fori_loop(0, n_chunks, one_chunk, 0)

    return pl.kernel(
        body,
        mesh=mesh,
        out_type=jax.ShapeDtypeStruct((n_rows, n_words), jnp.float32),
        scratch_types=[
            pltpu.VMEM((chunk,), jnp.int32),
            pltpu.VMEM((chunk, n_words), jnp.float32),
            pltpu.SemaphoreType.DMA,
        ],
    )(table, idx)


NC = 4
HCC = H // NC


def _fused_body(em_ref, nt_ref, x_ref, w1_ref, w2_ref, out_ref):
    t = pl.program_id(0)
    c = pl.program_id(1)

    @pl.when(t < nt_ref[0])
    def _():
        h = lax.dot_general(
            x_ref[...], w1_ref[0],
            (((1,), (1,)), ((), ())),
            preferred_element_type=jnp.float32,
        )
        part = lax.dot_general(
            h, w2_ref[0],
            (((1,), (1,)), ((), ())),
            preferred_element_type=jnp.float32,
        )

        @pl.when(c == 0)
        def _():
            out_ref[...] = part

        @pl.when(c > 0)
        def _():
            out_ref[...] += part


def _c_eff(t, c, nt):
    zig = jnp.where(t % 2 == 0, c, NC - 1 - c)
    frozen = jnp.where((nt - 1) % 2 == 0, NC - 1, 0)
    return jnp.where(t < nt, zig, frozen)


def _moe_fused(x_pad, w1, w2, expert_map, n_tiles):
    grid_spec = pltpu.PrefetchScalarGridSpec(
        num_scalar_prefetch=2,
        grid=(NTILES, NC),
        in_specs=[
            pl.BlockSpec(
                (T, D), lambda t, c, em, nt: (jnp.minimum(t, nt[0] - 1), 0)),
            pl.BlockSpec(
                (1, HCC, D),
                lambda t, c, em, nt: (em[t], _c_eff(t, c, nt[0]), 0)),
            pl.BlockSpec(
                (1, O, HCC),
                lambda t, c, em, nt: (em[t], 0, _c_eff(t, c, nt[0]))),
        ],
        out_specs=pl.BlockSpec((T, O), lambda t, c, em, nt: (t, 0)),
    )
    return pl.pallas_call(
        _fused_body,
        grid_spec=grid_spec,
        out_shape=jax.ShapeDtypeStruct((P, O), jnp.float32),
        compiler_params=pltpu.CompilerParams(
            vmem_limit_bytes=56 * 1024 * 1024),
    )(expert_map, n_tiles, x_pad, w1, w2)


def kernel(inp, gate, weight1, weight2):
    gate = gate.astype(jnp.int32)

    onehot = (gate[:, None] == jnp.arange(E, dtype=jnp.int32)[None, :])
    onehot_i = onehot.astype(jnp.int32)
    incl = jnp.cumsum(onehot_i, axis=0)
    counts = incl[-1]
    rank = jnp.sum(jnp.where(onehot, incl, 0), axis=1) - 1
    tiles_per_e = (counts + T - 1) // T
    tile_start = jnp.concatenate([jnp.zeros((1,), jnp.int32),
                                  jnp.cumsum(tiles_per_e)[:-1]]).astype(jnp.int32)
    n_tiles = tile_start[-1] + tiles_per_e[-1]
    pad_off = tile_start * T
    pad_pos = jnp.sum(jnp.where(onehot, pad_off[None, :], 0), axis=1) + rank
    pad_pos = pad_pos.astype(jnp.int32)
    t_ids = jnp.arange(NTILES, dtype=jnp.int32)
    expert_map = jnp.clip(
        jnp.sum((t_ids[:, None] >= tile_start[None, :]).astype(jnp.int32),
                axis=1) - 1, 0, E - 1).astype(jnp.int32)
    expert_map = expert_map[jnp.minimum(t_ids, n_tiles - 1)]

    idx3 = pad_pos.reshape(NW, 4, N // NW // 4)
    x_pad = _sc_scatter_rows(inp, idx3)

    nt = n_tiles.reshape(1)
    out_pad = _moe_fused(x_pad, weight1, weight2, expert_map, nt)

    return _sc_gather_rows(out_pad, pad_pos, 64)
